# baseline (speedup 1.0000x reference)
import functools

import jax
import jax.numpy as jnp
from jax import lax
from jax.experimental import pallas as pl
from jax.experimental.pallas import tpu as pltpu
from jax.experimental.pallas import tpu_sc as plsc

NC = 2
NS = 16
NW = NC * NS

CH = 80
WB = 624
ZR = 208
PBITS = 14
TRASH = 8


def _unpack(pidx, i, sid, did):
    def lane(j, _):
        v = pidx[i, pl.ds(j * 16, 16)]
        sid[0, pl.ds(j * 16, 16)] = jnp.bitwise_and(v, (1 << PBITS) - 1)
        did[0, pl.ds(j * 16, 16)] = jnp.right_shift(v, PBITS)
        return 0
    lax.fori_loop(0, CH // 16, lane, 0)


def _unpack_hi(pidx, i, did):
    def lane(j, _):
        v = pidx[i, pl.ds(j * 16, 16)]
        did[0, pl.ds(j * 16, 16)] = jnp.right_shift(v, PBITS)
        return 0
    lax.fori_loop(0, CH // 16, lane, 0)


def _zero_acc(z_hbm, acc, s, n):
    tail = n - NS * WB

    def zero(j, _):
        pltpu.sync_copy(z_hbm, acc.at[pl.ds(s * WB + j * ZR, ZR)])
        return 0

    lax.fori_loop(0, WB // ZR, zero, 0)

    @pl.when(s == NS - 1)
    def _():
        pltpu.sync_copy(z_hbm.at[pl.ds(0, tail)], acc.at[pl.ds(NS * WB, tail)])


def _writeback(acc, out_hbm, c, s, n):
    tail = n - NS * WB
    pltpu.sync_copy(acc.at[pl.ds(s * WB, WB)],
                    out_hbm.at[c, pl.ds(s * WB, WB)])

    @pl.when(s == NS - 1)
    def _():
        pltpu.sync_copy(acc.at[pl.ds(NS * WB, tail)],
                        out_hbm.at[c, pl.ds(NS * WB, tail)])


def _sc_degree(pk3, z_hbm, n):
    _, n_ch, _ = pk3.shape
    h = z_hbm.shape[1]
    mesh = plsc.VectorSubcoreMesh(core_axis_name="c", subcore_axis_name="s")

    @functools.partial(
        pl.kernel,
        out_type=jax.ShapeDtypeStruct((NC, n, h), jnp.float32),
        mesh=mesh,
        scratch_types=[
            pltpu.VMEM((n_ch, CH), jnp.int32),
            pltpu.VMEM((1, CH), jnp.int32),
            pltpu.VMEM((1, CH), jnp.int32),
            pltpu.VMEM((CH, h), jnp.float32),
            pltpu.VMEM_SHARED((n + TRASH, h), jnp.float32),
            pltpu.SemaphoreType.DMA,
            pltpu.SemaphoreType.DMA,
        ],
    )
    def deg_kernel(pk_hbm, zz_hbm, out_hbm, pidx, did0, did1, ones_v, acc,
                   sem0, sem1):
        c = lax.axis_index("c")
        s = lax.axis_index("s")
        w = c * NS + s

        pltpu.sync_copy(pk_hbm.at[w], pidx)

        def fill_ones(i, _):
            def fill_lane(j, _):
                ones_v[i, pl.ds(j * 16, 16)] = jnp.ones((16,), jnp.float32)
                return 0
            lax.fori_loop(0, h // 16, fill_lane, 0)
            return 0

        lax.fori_loop(0, CH, fill_ones, 0)
        _zero_acc(zz_hbm, acc, s, n)
        plsc.subcore_barrier()

        _unpack_hi(pidx, 0, did0)
        pltpu.async_copy(ones_v, acc.at[did0.at[0]], sem0, add=True)

        def step2(g, _):
            i1 = 2 * g + 1
            _unpack_hi(pidx, i1, did1)
            pltpu.async_copy(ones_v, acc.at[did1.at[0]], sem1, add=True)
            pltpu.make_async_copy(ones_v, acc.at[did0.at[0]], sem0).wait()

            @pl.when(i1 + 1 < n_ch)
            def _():
                _unpack_hi(pidx, i1 + 1, did0)
                pltpu.async_copy(ones_v, acc.at[did0.at[0]], sem0, add=True)

            pltpu.make_async_copy(ones_v, acc.at[did1.at[0]], sem1).wait()
            return 0

        lax.fori_loop(0, n_ch // 2, step2, 0)
        if n_ch % 2 == 1:
            pltpu.make_async_copy(ones_v, acc.at[did0.at[0]], sem0).wait()
        plsc.subcore_barrier()
        _writeback(acc, out_hbm, c, s, n)

    return deg_kernel(pk3, z_hbm)


def _sc_agg(hp, pk3, z_hbm):
    n, h = hp.shape
    _, n_ch, _ = pk3.shape
    mesh = plsc.VectorSubcoreMesh(core_axis_name="c", subcore_axis_name="s")

    @functools.partial(
        pl.kernel,
        out_type=jax.ShapeDtypeStruct((NC, n, h), jnp.float32),
        mesh=mesh,
        scratch_types=[
            pltpu.VMEM((n_ch, CH), jnp.int32),
            pltpu.VMEM((1, CH), jnp.int32),
            pltpu.VMEM((1, CH), jnp.int32),
            pltpu.VMEM((1, CH), jnp.int32),
            pltpu.VMEM((1, CH), jnp.int32),
            pltpu.VMEM((CH, h), jnp.float32),
            pltpu.VMEM((CH, h), jnp.float32),
            pltpu.VMEM_SHARED((n + TRASH, h), jnp.float32),
            pltpu.SemaphoreType.DMA,
            pltpu.SemaphoreType.DMA,
        ],
    )
    def agg_kernel(hp_hbm, pk_hbm, zz_hbm, out_hbm, pidx, sid0, did0, sid1,
                   did1, rows0, rows1, acc, gs0, gs1):
        c = lax.axis_index("c")
        s = lax.axis_index("s")
        w = c * NS + s

        pltpu.sync_copy(pk_hbm.at[w], pidx)
        _unpack(pidx, 0, sid0, did0)
        pltpu.async_copy(hp_hbm.at[sid0.at[0]], rows0, gs0)
        _zero_acc(zz_hbm, acc, s, n)
        plsc.subcore_barrier()

        def step2(g, _):
            i1 = 2 * g + 1
            _unpack(pidx, i1, sid1, did1)
            pltpu.make_async_copy(hp_hbm.at[sid0.at[0]], rows0, gs0).wait()
            pltpu.async_copy(hp_hbm.at[sid1.at[0]], rows1, gs1)
            pltpu.sync_copy(rows0, acc.at[did0.at[0]], add=True)

            @pl.when(i1 + 1 < n_ch)
            def _():
                _unpack(pidx, i1 + 1, sid0, did0)

            pltpu.make_async_copy(hp_hbm.at[sid1.at[0]], rows1, gs1).wait()

            @pl.when(i1 + 1 < n_ch)
            def _():
                pltpu.async_copy(hp_hbm.at[sid0.at[0]], rows0, gs0)

            pltpu.sync_copy(rows1, acc.at[did1.at[0]], add=True)
            return 0

        lax.fori_loop(0, n_ch // 2, step2, 0)
        if n_ch % 2 == 1:
            pltpu.make_async_copy(hp_hbm.at[sid0.at[0]], rows0, gs0).wait()
            pltpu.sync_copy(rows0, acc.at[did0.at[0]], add=True)
        plsc.subcore_barrier()
        _writeback(acc, out_hbm, c, s, n)

    return agg_kernel(hp, pk3, z_hbm)


_BLK = 1000
DW = 128


def _tc_mm(x, w1):
    n, d = x.shape
    h = w1.shape[1]

    def body(x_ref, w_ref, out_ref):
        out_ref[...] = jnp.dot(x_ref[...], w_ref[...],
                               preferred_element_type=jnp.float32)

    return pl.pallas_call(
        body,
        grid=(n // _BLK,),
        in_specs=[
            pl.BlockSpec((_BLK, d), lambda i: (i, 0)),
            pl.BlockSpec((d, h), lambda i: (0, 0)),
        ],
        out_specs=pl.BlockSpec((_BLK, h), lambda i: (i, 0)),
        out_shape=jax.ShapeDtypeStruct((n, h), jnp.float32),
    )(x, w1)


def _tc_scale(h1, degp):
    n, h = h1.shape

    def body(h_ref, deg_ref, hp_ref, dinv_ref):
        deg = deg_ref[0] + deg_ref[1] + 1.0
        dinv = lax.rsqrt(deg)[:, 0:1]
        hp_ref[...] = h_ref[...] * dinv
        dinv_ref[...] = dinv

    return pl.pallas_call(
        body,
        grid=(n // _BLK,),
        in_specs=[
            pl.BlockSpec((_BLK, h), lambda i: (i, 0)),
            pl.BlockSpec((NC, _BLK, DW), lambda i: (0, i, 0)),
        ],
        out_specs=[
            pl.BlockSpec((_BLK, h), lambda i: (i, 0)),
            pl.BlockSpec((_BLK, 1), lambda i: (i, 0)),
        ],
        out_shape=[
            jax.ShapeDtypeStruct((n, h), jnp.float32),
            jax.ShapeDtypeStruct((n, 1), jnp.float32),
        ],
    )(h1, degp)


def _tc_layer(agg, hp_prev, b_prev, dinv, w):
    n, h = hp_prev.shape
    hout = w.shape[1]
    grid = n // _BLK

    def body(agg_ref, hp_ref, b_ref, dinv_ref, w_ref, out_ref):
        xl = jnp.maximum(
            dinv_ref[...] * (agg_ref[0] + agg_ref[1] + hp_ref[...]) + b_ref[...],
            0.0)
        hmat = jnp.dot(xl, w_ref[...], preferred_element_type=jnp.float32)
        out_ref[...] = hmat * dinv_ref[...]

    return pl.pallas_call(
        body,
        grid=(grid,),
        in_specs=[
            pl.BlockSpec((NC, _BLK, h), lambda i: (0, i, 0)),
            pl.BlockSpec((_BLK, h), lambda i: (i, 0)),
            pl.BlockSpec((1, h), lambda i: (0, 0)),
            pl.BlockSpec((_BLK, 1), lambda i: (i, 0)),
            pl.BlockSpec((h, hout), lambda i: (0, 0)),
        ],
        out_specs=pl.BlockSpec((_BLK, hout), lambda i: (i, 0)),
        out_shape=jax.ShapeDtypeStruct((n, hout), jnp.float32),
    )(agg, hp_prev, b_prev.reshape(1, h), dinv, w)


def _tc_final(agg, hp3, b3, dinv, wc1, bc1, wc2, bc2):
    n, h = hp3.shape
    h1 = wc1.shape[1]
    grid = n // _BLK

    def body(agg_ref, hp_ref, b_ref, dinv_ref, wc1_ref, bc1_ref, wc2_ref,
             bc2_ref, out_ref, acc_ref):
        i = pl.program_id(0)

        @pl.when(i == 0)
        def _():
            acc_ref[...] = jnp.zeros_like(acc_ref)

        h3 = jnp.maximum(
            dinv_ref[...] * (agg_ref[0] + agg_ref[1] + hp_ref[...]) + b_ref[...],
            0.0)
        acc_ref[...] += jnp.sum(h3, axis=0, keepdims=True)

        @pl.when(i == grid - 1)
        def _():
            g = acc_ref[...] * (1.0 / n)
            z1 = jnp.maximum(
                jnp.dot(g, wc1_ref[...], preferred_element_type=jnp.float32)
                + bc1_ref[...], 0.0)
            z2 = (jnp.dot(z1, wc2_ref[...], preferred_element_type=jnp.float32)
                  + bc2_ref[...])
            m = jnp.max(z2, axis=1, keepdims=True)
            ez = jnp.exp(z2 - m)
            out_ref[...] = ez / jnp.sum(ez, axis=1, keepdims=True)

    return pl.pallas_call(
        body,
        grid=(grid,),
        in_specs=[
            pl.BlockSpec((NC, _BLK, h), lambda i: (0, i, 0)),
            pl.BlockSpec((_BLK, h), lambda i: (i, 0)),
            pl.BlockSpec((1, h), lambda i: (0, 0)),
            pl.BlockSpec((_BLK, 1), lambda i: (i, 0)),
            pl.BlockSpec(wc1.shape, lambda i: (0, 0)),
            pl.BlockSpec((1, h1), lambda i: (0, 0)),
            pl.BlockSpec(wc2.shape, lambda i: (0, 0)),
            pl.BlockSpec((1, 2), lambda i: (0, 0)),
        ],
        out_specs=pl.BlockSpec((1, 2), lambda i: (0, 0)),
        out_shape=jax.ShapeDtypeStruct((1, 2), jnp.float32),
        scratch_shapes=[pltpu.VMEM((1, h), jnp.float32)],
    )(agg, hp3, b3.reshape(1, h), dinv, wc1, bc1.reshape(1, h1), wc2,
      bc2.reshape(1, 2))


def kernel(x, edge_index, W1, b1, W2, b2, W3, b3, Wc1, bc1, Wc2, bc2):
    n = x.shape[0]
    e = edge_index.shape[1]
    per_w = e // NW
    n_ch = -(-per_w // CH)
    packed = jnp.bitwise_or(edge_index[0],
                            jnp.left_shift(edge_index[1], PBITS))
    pk2 = packed.reshape(NW, per_w)
    pk2 = jnp.pad(pk2, ((0, 0), (0, n_ch * CH - per_w)),
                  constant_values=n << PBITS)
    pk3 = pk2.reshape(NW, n_ch, CH)
    zeros = jnp.zeros((ZR, 128), jnp.float32)

    w3p = jnp.pad(W3, ((0, 0), (0, 128 - W3.shape[1])))
    b3p = jnp.pad(b3, (0, 128 - b3.shape[0]))
    wc1p = jnp.pad(Wc1, ((0, 128 - Wc1.shape[0]), (0, 0)))

    h1 = _tc_mm(x, W1)
    degp = _sc_degree(pk3, zeros, n)
    hp1, dinv = _tc_scale(h1, degp)
    agg1 = _sc_agg(hp1, pk3, zeros)
    hp2 = _tc_layer(agg1, hp1, b1, dinv, W2)
    agg2 = _sc_agg(hp2, pk3, zeros)
    hp3 = _tc_layer(agg2, hp2, b2, dinv, w3p)
    agg3 = _sc_agg(hp3, pk3, zeros)
    return _tc_final(agg3, hp3, b3p, dinv, wc1p, bc1, Wc2, bc2)


# speedup vs baseline: 1.0031x; 1.0031x over previous
import functools

import jax
import jax.numpy as jnp
from jax import lax
from jax.experimental import pallas as pl
from jax.experimental.pallas import tpu as pltpu
from jax.experimental.pallas import tpu_sc as plsc

NC = 2
NS = 16
NW = NC * NS

CH = 80
WB = 624
ZR = 208
PBITS = 14
TRASH = 8


def _unpack(pidx, i, sid, did):
    def lane(j, _):
        v = pidx[i, pl.ds(j * 16, 16)]
        sid[0, pl.ds(j * 16, 16)] = jnp.bitwise_and(v, (1 << PBITS) - 1)
        did[0, pl.ds(j * 16, 16)] = jnp.right_shift(v, PBITS)
        return 0
    lax.fori_loop(0, CH // 16, lane, 0)


def _unpack_hi(pidx, i, did):
    def lane(j, _):
        v = pidx[i, pl.ds(j * 16, 16)]
        did[0, pl.ds(j * 16, 16)] = jnp.right_shift(v, PBITS)
        return 0
    lax.fori_loop(0, CH // 16, lane, 0)


def _zero_acc(z_hbm, acc, s, n):
    tail = n - NS * WB

    def zero(j, _):
        pltpu.sync_copy(z_hbm, acc.at[pl.ds(s * WB + j * ZR, ZR)])
        return 0

    lax.fori_loop(0, WB // ZR, zero, 0)

    @pl.when(s == NS - 1)
    def _():
        pltpu.sync_copy(z_hbm.at[pl.ds(0, tail)], acc.at[pl.ds(NS * WB, tail)])


def _writeback(acc, out_hbm, c, s, n):
    tail = n - NS * WB
    pltpu.sync_copy(acc.at[pl.ds(s * WB, WB)],
                    out_hbm.at[c, pl.ds(s * WB, WB)])

    @pl.when(s == NS - 1)
    def _():
        pltpu.sync_copy(acc.at[pl.ds(NS * WB, tail)],
                        out_hbm.at[c, pl.ds(NS * WB, tail)])


def _sc_degree(pk3, z_hbm, n):
    _, n_ch, _ = pk3.shape
    h = z_hbm.shape[1]
    mesh = plsc.VectorSubcoreMesh(core_axis_name="c", subcore_axis_name="s")

    @functools.partial(
        pl.kernel,
        out_type=jax.ShapeDtypeStruct((NC, n, h), jnp.float32),
        mesh=mesh,
        scratch_types=[
            pltpu.VMEM((n_ch, CH), jnp.int32),
            pltpu.VMEM((1, CH), jnp.int32),
            pltpu.VMEM((1, CH), jnp.int32),
            pltpu.VMEM((CH, h), jnp.float32),
            pltpu.VMEM_SHARED((n + TRASH, h), jnp.float32),
            pltpu.SemaphoreType.DMA,
            pltpu.SemaphoreType.DMA,
        ],
    )
    def deg_kernel(pk_hbm, zz_hbm, out_hbm, pidx, did0, did1, ones_v, acc,
                   sem0, sem1):
        c = lax.axis_index("c")
        s = lax.axis_index("s")
        w = c * NS + s

        pltpu.sync_copy(pk_hbm.at[w], pidx)

        def fill_ones(i, _):
            def fill_lane(j, _):
                ones_v[i, pl.ds(j * 16, 16)] = jnp.ones((16,), jnp.float32)
                return 0
            lax.fori_loop(0, h // 16, fill_lane, 0)
            return 0

        lax.fori_loop(0, CH, fill_ones, 0)
        _zero_acc(zz_hbm, acc, s, n)
        plsc.subcore_barrier()

        _unpack_hi(pidx, 0, did0)
        pltpu.async_copy(ones_v, acc.at[did0.at[0]], sem0, add=True)

        def step2(g, _):
            i1 = 2 * g + 1
            _unpack_hi(pidx, i1, did1)
            pltpu.async_copy(ones_v, acc.at[did1.at[0]], sem1, add=True)
            pltpu.make_async_copy(ones_v, acc.at[did0.at[0]], sem0).wait()

            @pl.when(i1 + 1 < n_ch)
            def _():
                _unpack_hi(pidx, i1 + 1, did0)
                pltpu.async_copy(ones_v, acc.at[did0.at[0]], sem0, add=True)

            pltpu.make_async_copy(ones_v, acc.at[did1.at[0]], sem1).wait()
            return 0

        lax.fori_loop(0, n_ch // 2, step2, 0)
        if n_ch % 2 == 1:
            pltpu.make_async_copy(ones_v, acc.at[did0.at[0]], sem0).wait()
        plsc.subcore_barrier()
        _writeback(acc, out_hbm, c, s, n)

    return deg_kernel(pk3, z_hbm)


def _sc_agg(hp, pk3, z_hbm):
    n, h = hp.shape
    _, n_ch, _ = pk3.shape
    mesh = plsc.VectorSubcoreMesh(core_axis_name="c", subcore_axis_name="s")

    @functools.partial(
        pl.kernel,
        out_type=jax.ShapeDtypeStruct((NC, n, h), jnp.float32),
        mesh=mesh,
        scratch_types=[
            pltpu.VMEM((n_ch, CH), jnp.int32),
            pltpu.VMEM((1, CH), jnp.int32),
            pltpu.VMEM((1, CH), jnp.int32),
            pltpu.VMEM((1, CH), jnp.int32),
            pltpu.VMEM((1, CH), jnp.int32),
            pltpu.VMEM((CH, h), jnp.float32),
            pltpu.VMEM((CH, h), jnp.float32),
            pltpu.VMEM_SHARED((n + TRASH, h), jnp.float32),
            pltpu.SemaphoreType.DMA,
            pltpu.SemaphoreType.DMA,
        ],
    )
    def agg_kernel(hp_hbm, pk_hbm, zz_hbm, out_hbm, pidx, sid0, did0, sid1,
                   did1, rows0, rows1, acc, gs0, gs1):
        c = lax.axis_index("c")
        s = lax.axis_index("s")
        w = c * NS + s

        pltpu.sync_copy(pk_hbm.at[w], pidx)
        _unpack(pidx, 0, sid0, did0)
        pltpu.async_copy(hp_hbm.at[sid0.at[0]], rows0, gs0)
        _zero_acc(zz_hbm, acc, s, n)
        plsc.subcore_barrier()

        def step2(g, _):
            i1 = 2 * g + 1
            _unpack(pidx, i1, sid1, did1)
            pltpu.make_async_copy(hp_hbm.at[sid0.at[0]], rows0, gs0).wait()
            pltpu.async_copy(hp_hbm.at[sid1.at[0]], rows1, gs1)
            pltpu.sync_copy(rows0, acc.at[did0.at[0]], add=True)

            @pl.when(i1 + 1 < n_ch)
            def _():
                _unpack(pidx, i1 + 1, sid0, did0)

            pltpu.make_async_copy(hp_hbm.at[sid1.at[0]], rows1, gs1).wait()

            @pl.when(i1 + 1 < n_ch)
            def _():
                pltpu.async_copy(hp_hbm.at[sid0.at[0]], rows0, gs0)

            pltpu.sync_copy(rows1, acc.at[did1.at[0]], add=True)
            return 0

        lax.fori_loop(0, n_ch // 2, step2, 0)
        if n_ch % 2 == 1:
            pltpu.make_async_copy(hp_hbm.at[sid0.at[0]], rows0, gs0).wait()
            pltpu.sync_copy(rows0, acc.at[did0.at[0]], add=True)
        plsc.subcore_barrier()
        _writeback(acc, out_hbm, c, s, n)

    return agg_kernel(hp, pk3, z_hbm)


_BLK = 1000
DW = 128


def _tc_pre(x, w1, degp):
    n, d = x.shape
    h = w1.shape[1]
    grid = n // _BLK

    def body(x_ref, w_ref, deg_ref, hp_ref, dinv_ref):
        deg = deg_ref[0] + deg_ref[1] + 1.0
        dinv = lax.rsqrt(deg)[:, 0:1]
        hmat = jnp.dot(x_ref[...], w_ref[...], preferred_element_type=jnp.float32)
        hp_ref[...] = hmat * dinv
        dinv_ref[...] = dinv

    return pl.pallas_call(
        body,
        grid=(grid,),
        in_specs=[
            pl.BlockSpec((_BLK, d), lambda i: (i, 0)),
            pl.BlockSpec((d, h), lambda i: (0, 0)),
            pl.BlockSpec((NC, _BLK, DW), lambda i: (0, i, 0)),
        ],
        out_specs=[
            pl.BlockSpec((_BLK, h), lambda i: (i, 0)),
            pl.BlockSpec((_BLK, 1), lambda i: (i, 0)),
        ],
        out_shape=[
            jax.ShapeDtypeStruct((n, h), jnp.float32),
            jax.ShapeDtypeStruct((n, 1), jnp.float32),
        ],
    )(x, w1, degp)


def _tc_layer(agg, hp_prev, b_prev, dinv, w):
    n, h = hp_prev.shape
    hout = w.shape[1]
    grid = n // _BLK

    def body(agg_ref, hp_ref, b_ref, dinv_ref, w_ref, out_ref):
        xl = jnp.maximum(
            dinv_ref[...] * (agg_ref[0] + agg_ref[1] + hp_ref[...]) + b_ref[...],
            0.0)
        hmat = jnp.dot(xl, w_ref[...], preferred_element_type=jnp.float32)
        out_ref[...] = hmat * dinv_ref[...]

    return pl.pallas_call(
        body,
        grid=(grid,),
        in_specs=[
            pl.BlockSpec((NC, _BLK, h), lambda i: (0, i, 0)),
            pl.BlockSpec((_BLK, h), lambda i: (i, 0)),
            pl.BlockSpec((1, h), lambda i: (0, 0)),
            pl.BlockSpec((_BLK, 1), lambda i: (i, 0)),
            pl.BlockSpec((h, hout), lambda i: (0, 0)),
        ],
        out_specs=pl.BlockSpec((_BLK, hout), lambda i: (i, 0)),
        out_shape=jax.ShapeDtypeStruct((n, hout), jnp.float32),
    )(agg, hp_prev, b_prev.reshape(1, h), dinv, w)


def _tc_final(agg, hp3, b3, dinv, wc1, bc1, wc2, bc2):
    n, h = hp3.shape
    h1 = wc1.shape[1]
    grid = n // _BLK

    def body(agg_ref, hp_ref, b_ref, dinv_ref, wc1_ref, bc1_ref, wc2_ref,
             bc2_ref, out_ref, acc_ref):
        i = pl.program_id(0)

        @pl.when(i == 0)
        def _():
            acc_ref[...] = jnp.zeros_like(acc_ref)

        h3 = jnp.maximum(
            dinv_ref[...] * (agg_ref[0] + agg_ref[1] + hp_ref[...]) + b_ref[...],
            0.0)
        acc_ref[...] += jnp.sum(h3, axis=0, keepdims=True)

        @pl.when(i == grid - 1)
        def _():
            g = acc_ref[...] * (1.0 / n)
            z1 = jnp.maximum(
                jnp.dot(g, wc1_ref[...], preferred_element_type=jnp.float32)
                + bc1_ref[...], 0.0)
            z2 = (jnp.dot(z1, wc2_ref[...], preferred_element_type=jnp.float32)
                  + bc2_ref[...])
            m = jnp.max(z2, axis=1, keepdims=True)
            ez = jnp.exp(z2 - m)
            out_ref[...] = ez / jnp.sum(ez, axis=1, keepdims=True)

    return pl.pallas_call(
        body,
        grid=(grid,),
        in_specs=[
            pl.BlockSpec((NC, _BLK, h), lambda i: (0, i, 0)),
            pl.BlockSpec((_BLK, h), lambda i: (i, 0)),
            pl.BlockSpec((1, h), lambda i: (0, 0)),
            pl.BlockSpec((_BLK, 1), lambda i: (i, 0)),
            pl.BlockSpec(wc1.shape, lambda i: (0, 0)),
            pl.BlockSpec((1, h1), lambda i: (0, 0)),
            pl.BlockSpec(wc2.shape, lambda i: (0, 0)),
            pl.BlockSpec((1, 2), lambda i: (0, 0)),
        ],
        out_specs=pl.BlockSpec((1, 2), lambda i: (0, 0)),
        out_shape=jax.ShapeDtypeStruct((1, 2), jnp.float32),
        scratch_shapes=[pltpu.VMEM((1, h), jnp.float32)],
    )(agg, hp3, b3.reshape(1, h), dinv, wc1, bc1.reshape(1, h1), wc2,
      bc2.reshape(1, 2))


def kernel(x, edge_index, W1, b1, W2, b2, W3, b3, Wc1, bc1, Wc2, bc2):
    n = x.shape[0]
    e = edge_index.shape[1]
    per_w = e // NW
    n_ch = -(-per_w // CH)
    packed = jnp.bitwise_or(edge_index[0],
                            jnp.left_shift(edge_index[1], PBITS))
    pk2 = packed.reshape(NW, per_w)
    pk2 = jnp.pad(pk2, ((0, 0), (0, n_ch * CH - per_w)),
                  constant_values=n << PBITS)
    pk3 = pk2.reshape(NW, n_ch, CH)
    zeros = jnp.zeros((ZR, 128), jnp.float32)

    w3p = jnp.pad(W3, ((0, 0), (0, 128 - W3.shape[1])))
    b3p = jnp.pad(b3, (0, 128 - b3.shape[0]))
    wc1p = jnp.pad(Wc1, ((0, 128 - Wc1.shape[0]), (0, 0)))

    degp = _sc_degree(pk3, zeros, n)
    hp1, dinv = _tc_pre(x, W1, degp)
    agg1 = _sc_agg(hp1, pk3, zeros)
    hp2 = _tc_layer(agg1, hp1, b1, dinv, W2)
    agg2 = _sc_agg(hp2, pk3, zeros)
    hp3 = _tc_layer(agg2, hp2, b2, dinv, w3p)
    agg3 = _sc_agg(hp3, pk3, zeros)
    return _tc_final(agg3, hp3, b3p, dinv, wc1p, bc1, Wc2, bc2)
